# baseline (speedup 1.0000x reference)
import functools

import jax
import jax.numpy as jnp
from jax import lax
from jax.experimental import pallas as pl
from jax.experimental.pallas import tpu as pltpu
from jax.experimental.pallas import tpu_sc as plsc

EMB_DIM = 64
BATCH = 16384
NUM_CORES = 2
NUM_SUBCORES = 16
NW = NUM_CORES * NUM_SUBCORES
B_W = BATCH // NW
CHUNK = 128
NCH = B_W // CHUNK
LANES = 16
D_CH = EMB_DIM // LANES


PAD = 17


def _bpr_body(user_hbm, item_i_hbm, item_j_hbm, user_embs, item_embs,
              out_i_hbm, out_j_hbm,
              idx_u, idx_i, idx_j, u_rows, vi_rows, vj_rows,
              out_i_v, out_j_v, sem):
    wid = lax.axis_index("s") * NUM_CORES + lax.axis_index("c")
    base = wid * B_W

    pltpu.sync_copy(user_hbm.at[wid], idx_u)
    pltpu.sync_copy(item_i_hbm.at[wid], idx_i)
    pltpu.sync_copy(item_j_hbm.at[wid], idx_j)

    copies = []
    for c in range(NCH):
        dst = pl.ds(c * CHUNK, CHUNK)
        copies.append(pltpu.async_copy(user_embs.at[idx_u.at[c]],
                                       u_rows.at[dst], sem))
        copies.append(pltpu.async_copy(item_embs.at[idx_i.at[c]],
                                       vi_rows.at[dst], sem))
        copies.append(pltpu.async_copy(item_embs.at[idx_j.at[c]],
                                       vj_rows.at[dst], sem))
    for cp in copies:
        cp.wait()

    lane_iota = lax.iota(jnp.int32, LANES)
    perms = [lane_iota ^ s for s in (8, 4, 2, 1)]

    def lane_sum(v):
        for p in perms:
            v = v + v[p]
        return v

    def blk_body(g, carry):
        res_i = jnp.zeros((LANES,), jnp.float32)
        res_j = jnp.zeros((LANES,), jnp.float32)
        for k in range(LANES):
            r = g * LANES + k
            sl = pl.ds(0, LANES)
            u = u_rows[r, sl]
            acc_i = u * vi_rows[r, sl]
            acc_j = u * vj_rows[r, sl]
            for c in range(1, D_CH):
                sl = pl.ds(c * LANES, LANES)
                u = u_rows[r, sl]
                acc_i = acc_i + u * vi_rows[r, sl]
                acc_j = acc_j + u * vj_rows[r, sl]
            res_i = jnp.where(lane_iota == k, lane_sum(acc_i), res_i)
            res_j = jnp.where(lane_iota == k, lane_sum(acc_j), res_j)
        out_i_v[pl.ds(g * LANES, LANES)] = res_i
        out_j_v[pl.ds(g * LANES, LANES)] = res_j
        return carry

    lax.fori_loop(0, B_W // LANES, blk_body, 0)

    pltpu.sync_copy(out_i_v, out_i_hbm.at[pl.ds(base, B_W)])
    pltpu.sync_copy(out_j_v, out_j_hbm.at[pl.ds(base, B_W)])


@jax.jit
def _bpr(user_r, item_i_r, item_j_r, user_embs, item_embs):
    mesh = plsc.VectorSubcoreMesh(core_axis_name="c", subcore_axis_name="s")
    f = pl.kernel(
        _bpr_body,
        mesh=mesh,
        compiler_params=pltpu.CompilerParams(use_tc_tiling_on_sc=False),
        out_type=[
            jax.ShapeDtypeStruct((BATCH,), jnp.float32),
            jax.ShapeDtypeStruct((BATCH,), jnp.float32),
        ],
        scratch_types=[
            pltpu.VMEM((NCH, CHUNK), jnp.int32),
            pltpu.VMEM((NCH, CHUNK), jnp.int32),
            pltpu.VMEM((NCH, CHUNK), jnp.int32),
            pltpu.VMEM((B_W, EMB_DIM), jnp.float32),
            pltpu.VMEM((B_W, EMB_DIM), jnp.float32),
            pltpu.VMEM((B_W, EMB_DIM), jnp.float32),
            pltpu.VMEM((B_W,), jnp.float32),
            pltpu.VMEM((B_W,), jnp.float32),
            pltpu.SemaphoreType.DMA,
        ],
    )
    out_i, out_j = f(user_r, item_i_r, item_j_r, user_embs, item_embs)
    return out_i, out_j


def kernel(user, item_i, item_j, user_embs, item_embs):
    user_r = user.reshape(NW, NCH, CHUNK)
    item_i_r = item_i.reshape(NW, NCH, CHUNK)
    item_j_r = item_j.reshape(NW, NCH, CHUNK)
    return _bpr(user_r, item_i_r, item_j_r, user_embs, item_embs)
